# baseline (speedup 1.0000x reference)
import functools

import jax
import jax.numpy as jnp
from jax import lax
from jax.experimental import pallas as pl
from jax.experimental.pallas import tpu as pltpu
from jax.experimental.pallas import tpu_sc as plsc

N = 10000
E = 320000
GR = 128
NW = 32
DPW = 320
NPAD = NW * DPW
TRASH = DPW
ECHUNK = 2560
NJ = ECHUNK // 16
NPCHUNK = E // ECHUNK
SEG = 128
STAGE = ECHUNK + 16
CAP = 324608


def _mesh():
    return plsc.VectorSubcoreMesh(core_axis_name="c", subcore_axis_name="s")


def _worker_id():
    return lax.axis_index("s") * 2 + lax.axis_index("c")


def _partition_body(src_hbm, dst_hbm, ls_hbm, ld_hbm, cnt_hbm,
                    sbuf, dbuf, stage_s, stage_d, cvec):
    w = _worker_id()
    lo = w * DPW
    zero16 = jnp.zeros((16,), jnp.int32)
    trash16 = jnp.full((16,), TRASH, jnp.int32)

    def chunk_body(g, cnt):
        pltpu.sync_copy(dst_hbm.at[pl.ds(g * ECHUNK, ECHUNK)], dbuf)
        pltpu.sync_copy(src_hbm.at[pl.ds(g * ECHUNK, ECHUNK)], sbuf)

        def j_body(j, cl):
            vd = dbuf[pl.ds(j * 16, 16)]
            vs = sbuf[pl.ds(j * 16, 16)]
            m = (vd >= lo) & (vd < lo + DPW)
            pos = plsc.cumsum(jnp.where(m, 1, 0))
            idx = (cl - 1) + pos
            plsc.store_scatter(stage_s, [idx], vs, mask=m)
            plsc.store_scatter(stage_d, [idx], vd - lo, mask=m)
            return cl + pos[15]

        cl = lax.fori_loop(0, NJ, j_body, 0)
        stage_s[pl.ds(cl, 16)] = zero16
        stage_d[pl.ds(cl, 16)] = trash16
        off = pl.multiple_of(w * CAP + cnt, 16)
        pltpu.sync_copy(stage_s, ls_hbm.at[pl.ds(off, STAGE)])
        pltpu.sync_copy(stage_d, ld_hbm.at[pl.ds(off, STAGE)])
        return cnt + ((cl + 15) // 16) * 16

    cnt = lax.fori_loop(0, NPCHUNK, chunk_body, 0)

    def pad_body(j, _):
        stage_s[pl.ds(j * 16, 16)] = zero16
        stage_d[pl.ds(j * 16, 16)] = trash16
        return 0

    lax.fori_loop(0, 16, pad_body, 0)
    off = pl.multiple_of(w * CAP + cnt, 16)
    pltpu.sync_copy(stage_s.at[pl.ds(0, 256)], ls_hbm.at[pl.ds(off, 256)])
    pltpu.sync_copy(stage_d.at[pl.ds(0, 256)], ld_hbm.at[pl.ds(off, 256)])
    cvec[...] = jnp.full((16,), 0, jnp.int32) + cnt
    pltpu.sync_copy(cvec, cnt_hbm.at[pl.ds(pl.multiple_of(w * 16, 16), 16)])


def _partition(src, dst):
    f = pl.kernel(
        _partition_body,
        out_type=[
            jax.ShapeDtypeStruct((NW * CAP,), jnp.int32),
            jax.ShapeDtypeStruct((NW * CAP,), jnp.int32),
            jax.ShapeDtypeStruct((NW * 16,), jnp.int32),
        ],
        mesh=_mesh(),
        compiler_params=pltpu.CompilerParams(needs_layout_passes=False),
        scratch_types=[
            pltpu.VMEM((ECHUNK,), jnp.int32),
            pltpu.VMEM((ECHUNK,), jnp.int32),
            pltpu.VMEM((STAGE,), jnp.int32),
            pltpu.VMEM((STAGE,), jnp.int32),
            pltpu.VMEM((16,), jnp.int32),
        ],
    )
    return f(src, dst)


def _segmax_body(a_hbm, b_hbm, ls_hbm, ld_hbm, cnt_hbm, out_hbm,
                 acc, rows0, rows1, sidx0, sidx1, dloc0, dloc1, cvec,
                 sem0, sem1):
    w = _worker_id()
    lo = w * DPW
    neg = jnp.full((16,), float("-inf"), jnp.float32)

    def init_r(r, _):
        for v in range(8):
            acc[r, pl.ds(v * 16, 16)] = neg
        return 0

    lax.fori_loop(0, DPW + 1, init_r, 0)

    pltpu.sync_copy(cnt_hbm.at[pl.ds(pl.multiple_of(w * 16, 16), 16)], cvec)
    cnt = cvec[...][0]
    nsc = (cnt + 255) // 256

    def load_fire(g, sidx, dloc, rows, sem):
        off = pl.multiple_of(w * CAP + g * 256, 16)
        pltpu.sync_copy(ls_hbm.at[pl.ds(off, 128)], sidx.at[0])
        pltpu.sync_copy(ls_hbm.at[pl.ds(off + 128, 128)], sidx.at[1])
        pltpu.sync_copy(ld_hbm.at[pl.ds(off, 256)], dloc.at[pl.ds(0, 256)])
        pltpu.async_copy(b_hbm.at[sidx.at[0]], rows.at[pl.ds(0, 128)], sem)
        pltpu.async_copy(b_hbm.at[sidx.at[1]], rows.at[pl.ds(128, 128)], sem)

    def drain(sidx, rows, sem):
        pltpu.make_async_copy(b_hbm.at[sidx.at[0]], rows.at[pl.ds(0, 128)], sem).wait()
        pltpu.make_async_copy(b_hbm.at[sidx.at[1]], rows.at[pl.ds(128, 128)], sem).wait()

    def process(rows, dloc):
        def k_body(i, _):
            dv = dloc[pl.ds(i * 4, 16)]
            for u in range(4):
                dl = dv[u]
                k = i * 4 + u
                for v in range(8):
                    sl = pl.ds(v * 16, 16)
                    acc[dl, sl] = jnp.maximum(acc[dl, sl], rows[k, sl])
            return 0

        lax.fori_loop(0, 64, k_body, 0)

    @pl.when(nsc > 0)
    def _():
        load_fire(0, sidx0, dloc0, rows0, sem0)

    @pl.when(nsc > 1)
    def _():
        load_fire(1, sidx1, dloc1, rows1, sem1)

    def pair(p, _):
        g0 = 2 * p

        @pl.when(g0 < nsc)
        def _():
            drain(sidx0, rows0, sem0)
            process(rows0, dloc0)

            @pl.when(g0 + 2 < nsc)
            def _():
                load_fire(g0 + 2, sidx0, dloc0, rows0, sem0)

        g1 = g0 + 1

        @pl.when(g1 < nsc)
        def _():
            drain(sidx1, rows1, sem1)
            process(rows1, dloc1)

            @pl.when(g1 + 2 < nsc)
            def _():
                load_fire(g1 + 2, sidx1, dloc1, rows1, sem1)

        return 0

    lax.fori_loop(0, (nsc + 1) // 2, pair, 0)

    def fin_seg(rbuf, base, nrows):
        pltpu.sync_copy(
            a_hbm.at[pl.ds(pl.multiple_of(lo + base, 8), nrows)],
            rbuf.at[pl.ds(0, nrows)])

        def fin_r(i, _):
            for v in range(8):
                sl = pl.ds(v * 16, 16)
                acc[base + i, sl] = jnp.maximum(acc[base + i, sl] + rbuf[i, sl], 0.0)
            return 0

        lax.fori_loop(0, nrows, fin_r, 0)

    fin_seg(rows0, 0, 128)
    fin_seg(rows1, 128, 128)
    fin_seg(rows0, 256, 64)
    pltpu.sync_copy(acc.at[pl.ds(0, DPW)], out_hbm.at[pl.ds(pl.multiple_of(lo, 8), DPW)])


def _segmax(A, B, ls, ld, cnts):
    f = pl.kernel(
        _segmax_body,
        out_type=jax.ShapeDtypeStruct((NPAD, GR), jnp.float32),
        mesh=_mesh(),
        compiler_params=pltpu.CompilerParams(needs_layout_passes=False),
        scratch_types=[
            pltpu.VMEM((DPW + 1, GR), jnp.float32),
            pltpu.VMEM((256, GR), jnp.float32),
            pltpu.VMEM((256, GR), jnp.float32),
            pltpu.VMEM((2, 128), jnp.int32),
            pltpu.VMEM((2, 128), jnp.int32),
            pltpu.VMEM((272,), jnp.int32),
            pltpu.VMEM((272,), jnp.int32),
            pltpu.VMEM((16,), jnp.int32),
            pltpu.SemaphoreType.DMA,
            pltpu.SemaphoreType.DMA,
        ],
    )
    return f(A, B, ls, ld, cnts)


def _mm_block(h_ref, wd_ref, wb_ref, bias_ref, a_ref, b_ref):
    hb = h_ref[...]
    a_ref[...] = (jnp.dot(hb, wd_ref[...], preferred_element_type=jnp.float32)
                  + bias_ref[...])
    b_ref[...] = jnp.dot(hb, wb_ref[...], preferred_element_type=jnp.float32)


def _tc_mm(h, Wd, Wb, bias):
    M, cin = h.shape
    BM = 1024
    return pl.pallas_call(
        _mm_block,
        grid=(M // BM,),
        in_specs=[
            pl.BlockSpec((BM, cin), lambda i: (i, 0)),
            pl.BlockSpec((cin, GR), lambda i: (0, 0)),
            pl.BlockSpec((cin, GR), lambda i: (0, 0)),
            pl.BlockSpec((1, GR), lambda i: (0, 0)),
        ],
        out_specs=[
            pl.BlockSpec((BM, GR), lambda i: (i, 0)),
            pl.BlockSpec((BM, GR), lambda i: (i, 0)),
        ],
        out_shape=[
            jax.ShapeDtypeStruct((M, GR), jnp.float32),
            jax.ShapeDtypeStruct((M, GR), jnp.float32),
        ],
    )(h, Wd, Wb, bias)


def _final_block(x_ref, c0_ref, c1_ref, c2_ref, o_ref):
    for t, r in enumerate((x_ref, c0_ref, c1_ref, c2_ref)):
        v = r[...]
        p = jnp.maximum(jnp.maximum(v[:, 0:32], v[:, 32:64]),
                        jnp.maximum(v[:, 64:96], v[:, 96:128]))
        o_ref[:, t * 32:(t + 1) * 32] = p


def _tc_final(x, c0, c1, c2):
    BM = 2000
    return pl.pallas_call(
        _final_block,
        grid=(N // BM,),
        in_specs=[pl.BlockSpec((BM, GR), lambda i: (i, 0))] * 4,
        out_specs=pl.BlockSpec((BM, GR), lambda i: (i, 0)),
        out_shape=jax.ShapeDtypeStruct((N, GR), jnp.float32),
    )(x, c0, c1, c2)


_PERM = tuple(4 * (j % 32) + j // 32 for j in range(128))


def _permute_rows(Wpart):
    import numpy as np
    secs = [Wpart[0:GR]]
    perm = np.array(_PERM)
    for s in range(1, Wpart.shape[0] // GR):
        secs.append(Wpart[GR * s:GR * (s + 1)][perm])
    return jnp.concatenate(secs, axis=0)


def kernel(x, edge_index, W0, b0, W1, b1, W2, b2):
    import numpy as np
    perm = np.array(_PERM)
    src = edge_index[0].astype(jnp.int32)
    dst = edge_index[1].astype(jnp.int32)
    xp = jnp.pad(x, ((0, NPAD - N), (0, 0)))
    ls, ld, cnts = _partition(src, dst)
    h = xp
    cs = []
    for W, b in ((W0, b0), (W1, b1), (W2, b2)):
        cin = h.shape[1]
        Wd = _permute_rows((W[:cin] - W[cin:]))[:, perm]
        Wb = _permute_rows(W[cin:])[:, perm]
        A, B = _tc_mm(h, Wd, Wb, b[perm].reshape(1, GR))
        c = _segmax(A, B, ls, ld, cnts)
        cs.append(c)
        h = jnp.concatenate([h, c], axis=-1)
    return _tc_final(x[:, perm], cs[0], cs[1], cs[2])


# speedup vs baseline: 1.0120x; 1.0120x over previous
import functools

import jax
import jax.numpy as jnp
from jax import lax
from jax.experimental import pallas as pl
from jax.experimental.pallas import tpu as pltpu
from jax.experimental.pallas import tpu_sc as plsc

N = 10000
E = 320000
GR = 128
NW = 32
DPW = 320
NPAD = NW * DPW
TRASH = DPW
ECHUNK = 2560
NJ = ECHUNK // 16
NPCHUNK = E // ECHUNK
SEG = 128
NSH = 5120
STAGE = ECHUNK + 16
CAP = 324608


def _mesh():
    return plsc.VectorSubcoreMesh(core_axis_name="c", subcore_axis_name="s")


def _worker_id():
    return lax.axis_index("s") * 2 + lax.axis_index("c")


def _partition_body(src_hbm, dst_hbm, ls_hbm, ld_hbm, cnt_hbm,
                    sbuf, dbuf, stage_s, stage_d, cvec):
    w = _worker_id()
    lo = w * DPW
    zero16 = jnp.zeros((16,), jnp.int32)
    trash16 = jnp.full((16,), TRASH, jnp.int32)

    def chunk_body(g, cnt):
        pltpu.sync_copy(dst_hbm.at[pl.ds(g * ECHUNK, ECHUNK)], dbuf)
        pltpu.sync_copy(src_hbm.at[pl.ds(g * ECHUNK, ECHUNK)], sbuf)

        def j_body(j, cl):
            vd = dbuf[pl.ds(j * 16, 16)]
            vs = sbuf[pl.ds(j * 16, 16)]
            m = (vd >= lo) & (vd < lo + DPW)
            pos = plsc.cumsum(jnp.where(m, 1, 0))
            idx = (cl - 1) + pos
            plsc.store_scatter(stage_s, [idx], vs, mask=m)
            plsc.store_scatter(stage_d, [idx], vd - lo, mask=m)
            return cl + pos[15]

        cl = lax.fori_loop(0, NJ, j_body, 0)
        stage_s[pl.ds(cl, 16)] = zero16
        stage_d[pl.ds(cl, 16)] = trash16
        off = pl.multiple_of(w * CAP + cnt, 16)
        pltpu.sync_copy(stage_s, ls_hbm.at[pl.ds(off, STAGE)])
        pltpu.sync_copy(stage_d, ld_hbm.at[pl.ds(off, STAGE)])
        return cnt + ((cl + 15) // 16) * 16

    cnt = lax.fori_loop(0, NPCHUNK, chunk_body, 0)

    def pad_body(j, _):
        stage_s[pl.ds(j * 16, 16)] = zero16
        stage_d[pl.ds(j * 16, 16)] = trash16
        return 0

    lax.fori_loop(0, 16, pad_body, 0)
    off = pl.multiple_of(w * CAP + cnt, 16)
    pltpu.sync_copy(stage_s.at[pl.ds(0, 256)], ls_hbm.at[pl.ds(off, 256)])
    pltpu.sync_copy(stage_d.at[pl.ds(0, 256)], ld_hbm.at[pl.ds(off, 256)])
    cvec[...] = jnp.full((16,), 0, jnp.int32) + cnt
    pltpu.sync_copy(cvec, cnt_hbm.at[pl.ds(pl.multiple_of(w * 16, 16), 16)])


def _partition(src, dst):
    f = pl.kernel(
        _partition_body,
        out_type=[
            jax.ShapeDtypeStruct((NW * CAP,), jnp.int32),
            jax.ShapeDtypeStruct((NW * CAP,), jnp.int32),
            jax.ShapeDtypeStruct((NW * 16,), jnp.int32),
        ],
        mesh=_mesh(),
        compiler_params=pltpu.CompilerParams(needs_layout_passes=False),
        scratch_types=[
            pltpu.VMEM((ECHUNK,), jnp.int32),
            pltpu.VMEM((ECHUNK,), jnp.int32),
            pltpu.VMEM((STAGE,), jnp.int32),
            pltpu.VMEM((STAGE,), jnp.int32),
            pltpu.VMEM((16,), jnp.int32),
        ],
    )
    return f(src, dst)


def _segmax_body(b_hbm, ls_hbm, ld_hbm, cnt_hbm, out_hbm,
                 acc, rows0, rows1, sidx0, sidx1, dloc0, dloc1, cvec,
                 sem0, sem1):
    w = _worker_id()
    lo = w * DPW
    neg = jnp.full((16,), float("-inf"), jnp.float32)

    def init_r(r, _):
        for v in range(8):
            acc[r, pl.ds(v * 16, 16)] = neg
        return 0

    lax.fori_loop(0, DPW + 1, init_r, 0)

    pltpu.sync_copy(cnt_hbm.at[pl.ds(pl.multiple_of(w * 16, 16), 16)], cvec)
    cnt = cvec[...][0]
    nsc = (cnt + 255) // 256

    def load_fire(g, sidx, dloc, rows, sem):
        off = pl.multiple_of(w * CAP + g * 256, 16)
        pltpu.sync_copy(ls_hbm.at[pl.ds(off, 128)], sidx.at[0])
        pltpu.sync_copy(ls_hbm.at[pl.ds(off + 128, 128)], sidx.at[1])
        pltpu.sync_copy(ld_hbm.at[pl.ds(off, 256)], dloc.at[pl.ds(0, 256)])
        pltpu.async_copy(b_hbm.at[sidx.at[0]], rows.at[pl.ds(0, 128)], sem)
        pltpu.async_copy(b_hbm.at[sidx.at[1]], rows.at[pl.ds(128, 128)], sem)

    def drain(sidx, rows, sem):
        pltpu.make_async_copy(b_hbm.at[sidx.at[0]], rows.at[pl.ds(0, 128)], sem).wait()
        pltpu.make_async_copy(b_hbm.at[sidx.at[1]], rows.at[pl.ds(128, 128)], sem).wait()

    def process(rows, dloc):
        def k_body(i, _):
            dv = dloc[pl.ds(i * 4, 16)]
            for u in range(4):
                dl = dv[u]
                k = i * 4 + u
                for v in range(8):
                    sl = pl.ds(v * 16, 16)
                    acc[dl, sl] = jnp.maximum(acc[dl, sl], rows[k, sl])
            return 0

        lax.fori_loop(0, 64, k_body, 0)

    @pl.when(nsc > 0)
    def _():
        load_fire(0, sidx0, dloc0, rows0, sem0)

    @pl.when(nsc > 1)
    def _():
        load_fire(1, sidx1, dloc1, rows1, sem1)

    def pair(p, _):
        g0 = 2 * p

        @pl.when(g0 < nsc)
        def _():
            drain(sidx0, rows0, sem0)
            process(rows0, dloc0)

            @pl.when(g0 + 2 < nsc)
            def _():
                load_fire(g0 + 2, sidx0, dloc0, rows0, sem0)

        g1 = g0 + 1

        @pl.when(g1 < nsc)
        def _():
            drain(sidx1, rows1, sem1)
            process(rows1, dloc1)

            @pl.when(g1 + 2 < nsc)
            def _():
                load_fire(g1 + 2, sidx1, dloc1, rows1, sem1)

        return 0

    lax.fori_loop(0, (nsc + 1) // 2, pair, 0)

    pltpu.sync_copy(acc.at[pl.ds(0, DPW)], out_hbm.at[pl.ds(pl.multiple_of(lo, 8), DPW)])


def _segmax(B, ls, ld, cnts):
    f = pl.kernel(
        _segmax_body,
        out_type=jax.ShapeDtypeStruct((NPAD, GR), jnp.float32),
        mesh=_mesh(),
        compiler_params=pltpu.CompilerParams(needs_layout_passes=False),
        scratch_types=[
            pltpu.VMEM((DPW + 1, GR), jnp.float32),
            pltpu.VMEM((256, GR), jnp.float32),
            pltpu.VMEM((256, GR), jnp.float32),
            pltpu.VMEM((2, 128), jnp.int32),
            pltpu.VMEM((2, 128), jnp.int32),
            pltpu.VMEM((272,), jnp.int32),
            pltpu.VMEM((272,), jnp.int32),
            pltpu.VMEM((16,), jnp.int32),
            pltpu.SemaphoreType.DMA,
            pltpu.SemaphoreType.DMA,
        ],
    )
    return f(B, ls, ld, cnts)


def _mm_block(h_ref, wd_ref, wb_ref, bias_ref, a_ref, b_ref):
    hb = h_ref[...]
    a_ref[...] = (jnp.dot(hb, wd_ref[...], preferred_element_type=jnp.float32)
                  + bias_ref[...])
    b_ref[...] = jnp.dot(hb, wb_ref[...], preferred_element_type=jnp.float32)


def _tc_mm(h, Wd, Wb, bias):
    M, cin = h.shape
    BM = 1024
    return pl.pallas_call(
        _mm_block,
        grid=(M // BM,),
        in_specs=[
            pl.BlockSpec((BM, cin), lambda i: (i, 0)),
            pl.BlockSpec((cin, GR), lambda i: (0, 0)),
            pl.BlockSpec((cin, GR), lambda i: (0, 0)),
            pl.BlockSpec((1, GR), lambda i: (0, 0)),
        ],
        out_specs=[
            pl.BlockSpec((BM, GR), lambda i: (i, 0)),
            pl.BlockSpec((BM, GR), lambda i: (i, 0)),
        ],
        out_shape=[
            jax.ShapeDtypeStruct((M, GR), jnp.float32),
            jax.ShapeDtypeStruct((M, GR), jnp.float32),
        ],
    )(h, Wd, Wb, bias)


def _mm2_body(nparts):
    def body(*refs):
        part_refs = refs[:nparts]
        ap_ref, sp_ref, wd_ref, wb_ref, bias_ref, c_ref, a_ref, b_ref = refs[nparts:]
        c = jnp.maximum(ap_ref[...].astype(jnp.float32)
                        + sp_ref[...].astype(jnp.float32), 0.0)
        c_ref[...] = c
        acc_a = jnp.zeros(c.shape, jnp.float32)
        acc_b = jnp.zeros(c.shape, jnp.float32)
        for j in range(nparts + 1):
            p = part_refs[j][...] if j < nparts else c
            sl = slice(GR * j, GR * (j + 1))
            acc_a += jnp.dot(p, wd_ref[sl, :], preferred_element_type=jnp.float32)
            acc_b += jnp.dot(p, wb_ref[sl, :], preferred_element_type=jnp.float32)
        a_ref[...] = acc_a + bias_ref[...]
        b_ref[...] = acc_b
    return body


def _tc_mm2(parts, A_prev, S_prev, Wd, Wb, bias):
    M = parts[0].shape[0]
    cin = GR * (len(parts) + 1)
    BM = 1024
    return pl.pallas_call(
        _mm2_body(len(parts)),
        grid=(M // BM,),
        in_specs=[pl.BlockSpec((BM, GR), lambda i: (i, 0))] * len(parts) + [
            pl.BlockSpec((BM, GR), lambda i: (i, 0)),
            pl.BlockSpec((BM, GR), lambda i: (i, 0)),
            pl.BlockSpec((cin, GR), lambda i: (0, 0)),
            pl.BlockSpec((cin, GR), lambda i: (0, 0)),
            pl.BlockSpec((1, GR), lambda i: (0, 0)),
        ],
        out_specs=[
            pl.BlockSpec((BM, GR), lambda i: (i, 0)),
            pl.BlockSpec((BM, GR), lambda i: (i, 0)),
            pl.BlockSpec((BM, GR), lambda i: (i, 0)),
        ],
        out_shape=[
            jax.ShapeDtypeStruct((M, GR), jnp.float32),
            jax.ShapeDtypeStruct((M, GR), jnp.float32),
            jax.ShapeDtypeStruct((M, GR), jnp.float32),
        ],
    )(*parts, A_prev, S_prev, Wd, Wb, bias)


def _final_block(x_ref, c0_ref, c1_ref, a2_ref, s2_ref, o_ref):
    c2 = jnp.maximum(a2_ref[...].astype(jnp.float32)
                     + s2_ref[...].astype(jnp.float32), 0.0)
    for t, v in enumerate((x_ref[...], c0_ref[...], c1_ref[...], c2)):
        p = jnp.maximum(jnp.maximum(v[:, 0:32], v[:, 32:64]),
                        jnp.maximum(v[:, 64:96], v[:, 96:128]))
        o_ref[:, t * 32:(t + 1) * 32] = p.astype(jnp.float32)


def _tc_final(x, c0, c1, A2, S2):
    BM = 2000
    return pl.pallas_call(
        _final_block,
        grid=(N // BM,),
        in_specs=[pl.BlockSpec((BM, GR), lambda i: (i, 0))] * 5,
        out_specs=pl.BlockSpec((BM, GR), lambda i: (i, 0)),
        out_shape=jax.ShapeDtypeStruct((N, GR), jnp.float32),
    )(x, c0, c1, A2, S2)


_PERM = tuple(4 * (j % 32) + j // 32 for j in range(128))


def _permute_rows(Wpart):
    import numpy as np
    secs = [Wpart[0:GR]]
    perm = np.array(_PERM)
    for s in range(1, Wpart.shape[0] // GR):
        secs.append(Wpart[GR * s:GR * (s + 1)][perm])
    return jnp.concatenate(secs, axis=0)


def kernel(x, edge_index, W0, b0, W1, b1, W2, b2):
    import numpy as np
    perm = np.array(_PERM)
    src = edge_index[0].astype(jnp.int32)
    dst = edge_index[1].astype(jnp.int32)
    xp = jnp.pad(x, ((0, NPAD - N), (0, 0)))
    ls, ld, cnts = _partition(src, dst)

    def prep(W, b, cin):
        Wd = _permute_rows((W[:cin] - W[cin:]))[:, perm]
        Wb = _permute_rows(W[cin:])[:, perm]
        return Wd, Wb, b[perm].reshape(1, GR)

    Wd0, Wb0, bias0 = prep(W0, b0, GR)
    A0, B0 = _tc_mm(xp, Wd0, Wb0, bias0)
    S0 = _segmax(B0, ls, ld, cnts)

    Wd1, Wb1, bias1 = prep(W1, b1, 2 * GR)
    c0, A1, B1 = _tc_mm2([xp], A0, S0, Wd1, Wb1, bias1)
    S1 = _segmax(B1, ls, ld, cnts)

    Wd2, Wb2, bias2 = prep(W2, b2, 3 * GR)
    c1, A2, B2 = _tc_mm2([xp, c0], A1, S1, Wd2, Wb2, bias2)
    S2 = _segmax(B2, ls, ld, cnts)

    return _tc_final(x[:, perm], c0, c1, A2, S2)
